# baseline (device time: 1256225 ns/iter reference)
import jax
import jax.numpy as jnp
from jax import lax
from jax.experimental import pallas as pl
from jax.experimental.pallas import tpu as pltpu

N_DEV = 4
M, N = 4096, 8192
N_PIPES = 8
PIPE_ROWS = M // N_PIPES
ROWS = 32
ROUNDS = PIPE_ROWS // (N_DEV * ROWS)
STEPS = 2 * (N_DEV - 1)
TOTAL = ROUNDS * STEPS
PIPE_DIR = (1, 1, 1, 1, -1, -1, -1, -1)
PIPE_BASE = tuple(i * PIPE_ROWS for i in range(N_PIPES))
PIPE_ORDER = (0, 4, 1, 5, 2, 6, 3, 7)


def _silu(y):
    return y * (1.0 / (1.0 + jnp.exp(-y)))


def kernel(x, w_mat):
    p = jnp.dot(x, w_mat, preferred_element_type=jnp.float32)

    def body(p_ref, o_ref, *scr):
        P = N_PIPES
        comms = scr[0:P]
        locs = scr[P:2 * P]
        ssems = scr[2 * P:3 * P]
        rsems = scr[3 * P:4 * P]
        csems = scr[4 * P:5 * P]
        stsems = scr[5 * P:6 * P]
        creds = scr[6 * P:7 * P]

        my = lax.axis_index("i")
        left = jnp.mod(my - 1, N_DEV)
        right = jnp.mod(my + 1, N_DEV)

        barrier = pltpu.get_barrier_semaphore()
        for nbr in (left, right):
            pl.semaphore_signal(barrier, inc=1, device_id=(nbr,),
                                device_id_type=pl.DeviceIdType.MESH)
        pl.semaphore_wait(barrier, 2)

        def tgt(pi):
            return right if PIPE_DIR[pi] > 0 else left

        def src(pi):
            return left if PIPE_DIR[pi] > 0 else right

        def rchunk(pi, u):
            t = u % STEPS
            return jnp.mod(my - PIPE_DIR[pi] * (t + 1), N_DEV)

        def goff(pi, u):
            return PIPE_BASE[pi] + ((u // STEPS) * N_DEV + rchunk(pi, u)) * ROWS

        def mk_rdma(pi, u):
            s_slot, r_slot = u % 2, (u + 1) % 2
            return pltpu.make_async_remote_copy(
                src_ref=comms[pi].at[s_slot],
                dst_ref=comms[pi].at[r_slot],
                send_sem=ssems[pi].at[s_slot],
                recv_sem=rsems[pi].at[r_slot],
                device_id=(tgt(pi),),
                device_id_type=pl.DeviceIdType.MESH,
            )

        def start_load(pi, dst, off):
            c = pltpu.make_async_copy(p_ref.at[pl.ds(off, ROWS)], dst,
                                      csems[pi])
            c.start()
            return c

        init = [start_load(pi, comms[pi].at[0], PIPE_BASE[pi] + my * ROWS)
                for pi in range(N_PIPES)]
        rdmas = [None] * N_PIPES
        loads = [None] * N_PIPES
        stores = [None] * N_PIPES
        for pi in range(N_PIPES):
            init[pi].wait()
            rdmas[pi] = mk_rdma(pi, 0)
            rdmas[pi].start()
            loads[pi] = start_load(pi, locs[pi], goff(pi, 0))

        for u in range(TOTAL):
            t = u % STEPS
            g = u // STEPS
            r_slot = (u + 1) % 2
            for pi in PIPE_ORDER:
                rdmas[pi].wait()
                if stores[pi] is not None:
                    stores[pi].wait()
                    stores[pi] = None
                if u < TOTAL - 1:
                    pl.semaphore_signal(creds[pi], inc=1,
                                        device_id=(src(pi),),
                                        device_id_type=pl.DeviceIdType.MESH)
                off = goff(pi, u)
                if t < N_DEV - 1:
                    loads[pi].wait()
                    loads[pi] = None
                    acc = comms[pi][r_slot] + locs[pi][...]
                    if t == N_DEV - 2:
                        acc = _silu(acc)
                    comms[pi][r_slot] = acc
                if t >= N_DEV - 2:
                    stc = pltpu.make_async_copy(
                        comms[pi].at[r_slot],
                        o_ref.at[pl.ds(off, ROWS)], stsems[pi])
                    stc.start()
                    stores[pi] = stc
                if u + 1 < TOTAL:
                    t2 = (u + 1) % STEPS
                    if t2 == 0:
                        if stores[pi] is not None:
                            stores[pi].wait()
                            stores[pi] = None
                        c0 = start_load(
                            pi, comms[pi].at[0],
                            PIPE_BASE[pi] + ((g + 1) * N_DEV + my) * ROWS)
                        c0.wait()
                    pl.semaphore_wait(creds[pi], 1)
                    rdmas[pi] = mk_rdma(pi, u + 1)
                    rdmas[pi].start()
                    if t2 < N_DEV - 1:
                        loads[pi] = start_load(pi, locs[pi], goff(pi, u + 1))

        for pi in PIPE_ORDER:
            if stores[pi] is not None:
                stores[pi].wait()

    return pl.pallas_call(
        body,
        out_shape=jax.ShapeDtypeStruct((M, N), jnp.float32),
        in_specs=[pl.BlockSpec(memory_space=pl.ANY)],
        out_specs=pl.BlockSpec(memory_space=pl.ANY),
        scratch_shapes=(
            [pltpu.VMEM((2, ROWS, N), jnp.float32)] * N_PIPES
            + [pltpu.VMEM((ROWS, N), jnp.float32)] * N_PIPES
            + [pltpu.SemaphoreType.DMA((2,))] * N_PIPES
            + [pltpu.SemaphoreType.DMA((2,))] * N_PIPES
            + [pltpu.SemaphoreType.DMA] * N_PIPES
            + [pltpu.SemaphoreType.DMA] * N_PIPES
            + [pltpu.SemaphoreType.REGULAR] * N_PIPES
        ),
        compiler_params=pltpu.CompilerParams(collective_id=0),
    )(p)
